# baseline (speedup 1.0000x reference)
import numpy as np
import jax
import jax.numpy as jnp
from jax.experimental import pallas as pl
from jax.experimental.pallas import tpu as pltpu

_MUS32 = np.minimum(2.0 / 20 / 2.0 - 1.0 + (2.0 / 20) * np.arange(21), 1.0).astype(np.float32)
_MU64 = _MUS32.astype(np.float64)
_A = 1.0 / (2.0 * np.float64(np.float32(0.1)) ** 2)
_AE = 1.0 / (2.0 * np.float64(np.float32(0.001)) ** 2)
_K0 = 10
_UA = np.float32(_A)
_U1 = np.float32(2.0 * _A * _MU64[_K0])
_TS = np.float32(2.0 * _A * 0.1)
_CROW = np.concatenate([np.exp(-_A * _MU64[:20] ** 2), [1.0]]).astype(np.float32)
_CE = np.float32(_AE)

_TILE_B = 8
_NORM_ROWS = 1000
_UNROLL = 64

_CompilerParams = getattr(pltpu, "CompilerParams", None) or getattr(pltpu, "TPUCompilerParams")
_MemSpace = getattr(pltpu, "MemorySpace", None) or getattr(pltpu, "TPUMemorySpace")
_SMEM = _MemSpace.SMEM


def _norm_body(e_ref, o_ref):
    x = e_ref[...]
    n = jnp.sqrt(jnp.sum(x * x, axis=-1, keepdims=True))
    o_ref[...] = x / jnp.maximum(n, 1e-12)


def _feats_one(q, d, c_row):
    mm = jax.lax.dot_general(
        q, d, (((1,), (1,)), ((), ())), preferred_element_type=jnp.float32
    )
    base = jnp.exp(mm * (_U1 - _UA * mm))
    t = jnp.exp(_TS * mm)
    tinv = 1.0 / t
    sums = [None] * 21
    p = base
    sums[_K0] = jnp.sum(p, axis=-1, keepdims=True)
    for k in range(_K0 + 1, 20):
        p = p * t
        sums[k] = jnp.sum(p, axis=-1, keepdims=True)
    p = base
    for k in range(_K0 - 1, -1, -1):
        p = p * tinv
        sums[k] = jnp.sum(p, axis=-1, keepdims=True)
    dm = mm - 1.0
    sums[20] = jnp.sum(jnp.exp(dm * dm * (-_CE)), axis=-1, keepdims=True)
    s = jnp.concatenate(sums, axis=1) * c_row
    return jnp.sum(jnp.log1p(s), axis=0, keepdims=True)


_CHUNK = 128


def _knrm_body(qi1_ref, di1_ref, qi2_ref, di2_ref, en_ref, w1_ref, b1_ref,
               w2_ref, b2_ref, w3_ref, b3_ref, c_ref, out_ref, qbuf, dbuf):
    tb = qi1_ref.shape[0]
    Q = qi1_ref.shape[1]
    D = di1_ref.shape[1]
    c_row = c_ref[...]
    feats = []
    for ids_q, ids_d in ((qi1_ref, di1_ref), (qi2_ref, di2_ref)):
        for i in range(tb):
            for m in range(Q):
                qbuf[m] = en_ref[ids_q[i, m], 0]
            qv = qbuf[...]

            def chunk(g, S, i=i, ids_d=ids_d, qv=qv):
                base = g * _CHUNK
                for j in range(_CHUNK):
                    dbuf[j] = en_ref[ids_d[i, base + j], 0]
                mm = jax.lax.dot_general(
                    qv, dbuf[0:_CHUNK, :], (((1,), (1,)), ((), ())),
                    preferred_element_type=jnp.float32)
                basev = jnp.exp(mm * (_U1 - _UA * mm))
                t = jnp.exp(_TS * mm)
                tinv = 1.0 / t
                sums = [None] * 21
                p = basev
                sums[_K0] = jnp.sum(p, axis=-1, keepdims=True)
                for k in range(_K0 + 1, 20):
                    p = p * t
                    sums[k] = jnp.sum(p, axis=-1, keepdims=True)
                p = basev
                for k in range(_K0 - 1, -1, -1):
                    p = p * tinv
                    sums[k] = jnp.sum(p, axis=-1, keepdims=True)
                dm = mm - 1.0
                sums[20] = jnp.sum(jnp.exp(dm * dm * (-_CE)), axis=-1, keepdims=True)
                return S + jnp.concatenate(sums, axis=1)

            S = jax.lax.fori_loop(0, D // _CHUNK, chunk,
                                  jnp.zeros((Q, 21), jnp.float32))
            feats.append(jnp.sum(jnp.log1p(S * c_row), axis=0, keepdims=True))
    f = jnp.concatenate(feats, axis=0)
    h = jnp.maximum(jnp.dot(f, w1_ref[...]) + b1_ref[...], 0.0)
    h = jnp.maximum(jnp.dot(h, w2_ref[...]) + b2_ref[...], 0.0)
    lg = jnp.dot(h, w3_ref[...]) + b3_ref[...]
    out_ref[...] = 1.0 / (1.0 + jnp.exp(lg[tb:] - lg[:tb]))


def kernel(query_1, document_1, query_2, document_2, E, W1, b1, W2, b2, W3, b3):
    V, EMB = E.shape
    B, Q = query_1.shape
    D = document_1.shape[1]

    En = pl.pallas_call(
        _norm_body,
        grid=(V // _NORM_ROWS,),
        in_specs=[pl.BlockSpec((_NORM_ROWS, 1, EMB), lambda i: (i, 0, 0))],
        out_specs=pl.BlockSpec((_NORM_ROWS, 1, EMB), lambda i: (i, 0, 0)),
        out_shape=jax.ShapeDtypeStruct((V, 1, EMB), jnp.float32),
        compiler_params=_CompilerParams(dimension_semantics=("parallel",)),
    )(E.astype(jnp.float32).reshape(V, 1, EMB))

    def smem_ids(cols):
        return pl.BlockSpec((_TILE_B, cols), lambda i: (i, 0), memory_space=_SMEM)

    def block2(shape):
        return pl.BlockSpec(shape, lambda i: (0, 0))

    out = pl.pallas_call(
        _knrm_body,
        grid=(B // _TILE_B,),
        in_specs=[
            smem_ids(Q),
            smem_ids(D),
            smem_ids(Q),
            smem_ids(D),
            pl.BlockSpec((V, 1, EMB), lambda i: (0, 0, 0)),
            block2(W1.shape),
            block2((1, b1.shape[0])),
            block2(W2.shape),
            block2((1, b2.shape[0])),
            block2(W3.shape),
            block2((1, b3.shape[0])),
            block2((1, 21)),
        ],
        out_specs=pl.BlockSpec((_TILE_B, 1), lambda i: (i, 0)),
        out_shape=jax.ShapeDtypeStruct((B, 1), jnp.float32),
        scratch_shapes=[
            pltpu.VMEM((Q, EMB), jnp.float32),
            pltpu.VMEM((D, EMB), jnp.float32),
        ],
        compiler_params=_CompilerParams(dimension_semantics=("parallel",)),
    )(query_1, document_1, query_2, document_2, En, W1, b1.reshape(1, -1),
      W2, b2.reshape(1, -1), W3, b3.reshape(1, -1), jnp.asarray(_CROW).reshape(1, 21))
    return out


# speedup vs baseline: 1.7781x; 1.7781x over previous
import numpy as np
import jax
import jax.numpy as jnp
from jax.experimental import pallas as pl
from jax.experimental.pallas import tpu as pltpu

_MUS32 = np.minimum(2.0 / 20 / 2.0 - 1.0 + (2.0 / 20) * np.arange(21), 1.0).astype(np.float32)
_MU64 = _MUS32.astype(np.float64)
_A = 1.0 / (2.0 * np.float64(np.float32(0.1)) ** 2)
_AE = 1.0 / (2.0 * np.float64(np.float32(0.001)) ** 2)
_K0 = 10
_UA = np.float32(_A)
_U1 = np.float32(2.0 * _A * _MU64[_K0])
_TS = np.float32(2.0 * _A * 0.1)
_CROW = np.concatenate([np.exp(-_A * _MU64[:20] ** 2), [1.0]]).astype(np.float32)
_CE = np.float32(_AE)

_TILE_B = 8
_NORM_ROWS = 1000
_UNROLL = 64

_CompilerParams = getattr(pltpu, "CompilerParams", None) or getattr(pltpu, "TPUCompilerParams")
_MemSpace = getattr(pltpu, "MemorySpace", None) or getattr(pltpu, "TPUMemorySpace")
_SMEM = _MemSpace.SMEM


def _norm_body(e_ref, o_ref):
    x = e_ref[...]
    n = jnp.sqrt(jnp.sum(x * x, axis=-1, keepdims=True))
    o_ref[...] = x / jnp.maximum(n, 1e-12)


def _feats_one(q, d, c_row):
    mm = jax.lax.dot_general(
        q, d, (((1,), (1,)), ((), ())), preferred_element_type=jnp.float32
    )
    base = jnp.exp(mm * (_U1 - _UA * mm))
    t = jnp.exp(_TS * mm)
    tinv = 1.0 / t
    sums = [None] * 21
    p = base
    sums[_K0] = jnp.sum(p, axis=-1, keepdims=True)
    for k in range(_K0 + 1, 20):
        p = p * t
        sums[k] = jnp.sum(p, axis=-1, keepdims=True)
    p = base
    for k in range(_K0 - 1, -1, -1):
        p = p * tinv
        sums[k] = jnp.sum(p, axis=-1, keepdims=True)
    dm = mm - 1.0
    sums[20] = jnp.sum(jnp.exp(dm * dm * (-_CE)), axis=-1, keepdims=True)
    s = jnp.concatenate(sums, axis=1) * c_row
    return jnp.sum(jnp.log1p(s), axis=0, keepdims=True)


def _knrm_body(qi1_ref, di1_ref, qi2_ref, di2_ref, en_ref, w1_ref, b1_ref,
               w2_ref, b2_ref, w3_ref, b3_ref, c_ref, out_ref, qbuf, dbuf):
    tb = qi1_ref.shape[0]
    Q = qi1_ref.shape[1]
    D = di1_ref.shape[1]
    c_row = c_ref[...]
    feats = []
    for ids_q, ids_d in ((qi1_ref, di1_ref), (qi2_ref, di2_ref)):
        for i in range(tb):
            for m in range(Q):
                qbuf[m] = en_ref[ids_q[i, m], 0]

            def gath(g, _, i=i, ids_d=ids_d):
                base = g * _UNROLL
                for j in range(_UNROLL):
                    dbuf[base + j] = en_ref[ids_d[i, base + j], 0]
                return 0

            jax.lax.fori_loop(0, D // _UNROLL, gath, 0)
            feats.append(_feats_one(qbuf[...], dbuf[...], c_row))
    f = jnp.concatenate(feats, axis=0)
    h = jnp.maximum(jnp.dot(f, w1_ref[...]) + b1_ref[...], 0.0)
    h = jnp.maximum(jnp.dot(h, w2_ref[...]) + b2_ref[...], 0.0)
    lg = jnp.dot(h, w3_ref[...]) + b3_ref[...]
    out_ref[...] = 1.0 / (1.0 + jnp.exp(lg[tb:] - lg[:tb]))


def kernel(query_1, document_1, query_2, document_2, E, W1, b1, W2, b2, W3, b3):
    V, EMB = E.shape
    B, Q = query_1.shape
    D = document_1.shape[1]

    En = pl.pallas_call(
        _norm_body,
        grid=(V // _NORM_ROWS,),
        in_specs=[pl.BlockSpec((_NORM_ROWS, 1, EMB), lambda i: (i, 0, 0))],
        out_specs=pl.BlockSpec((_NORM_ROWS, 1, EMB), lambda i: (i, 0, 0)),
        out_shape=jax.ShapeDtypeStruct((V, 1, EMB), jnp.float32),
        compiler_params=_CompilerParams(dimension_semantics=("parallel",)),
    )(E.astype(jnp.float32).reshape(V, 1, EMB))

    def smem_ids(cols):
        return pl.BlockSpec((_TILE_B, cols), lambda i: (i, 0), memory_space=_SMEM)

    def block2(shape):
        return pl.BlockSpec(shape, lambda i: (0, 0))

    out = pl.pallas_call(
        _knrm_body,
        grid=(B // _TILE_B,),
        in_specs=[
            smem_ids(Q),
            smem_ids(D),
            smem_ids(Q),
            smem_ids(D),
            pl.BlockSpec((V, 1, EMB), lambda i: (0, 0, 0)),
            block2(W1.shape),
            block2((1, b1.shape[0])),
            block2(W2.shape),
            block2((1, b2.shape[0])),
            block2(W3.shape),
            block2((1, b3.shape[0])),
            block2((1, 21)),
        ],
        out_specs=pl.BlockSpec((_TILE_B, 1), lambda i: (i, 0)),
        out_shape=jax.ShapeDtypeStruct((B, 1), jnp.float32),
        scratch_shapes=[
            pltpu.VMEM((Q, EMB), jnp.float32),
            pltpu.VMEM((D, EMB), jnp.float32),
        ],
        compiler_params=_CompilerParams(dimension_semantics=("parallel",)),
    )(query_1, document_1, query_2, document_2, En, W1, b1.reshape(1, -1),
      W2, b2.reshape(1, -1), W3, b3.reshape(1, -1), jnp.asarray(_CROW).reshape(1, 21))
    return out
